# baseline (speedup 1.0000x reference)
import functools

import jax
import jax.numpy as jnp
from jax import lax
from jax.experimental import pallas as pl
from jax.experimental.pallas import tpu as pltpu
from jax.experimental.pallas import tpu_sc as plsc

B, L = 4096, 64
MOD_IN = 109
K = 6
MOD_OUT = 8
OUT_FEATURES = 256
AA_DIM = OUT_FEATURES - MOD_OUT
VOCAB = 128

B1 = 2048
B2 = B - B1
RB = 256
R = RB * L

NTOK2 = B2 * L
NC, NS = 2, 16
NW = NC * NS
PERW = NTOK2 // NW
C = 256
NCHUNK = PERW // C


def _tc_full_body(idx_ref, mod_ref, wa_ref, wb_ref, out_ref):
    idx3 = idx_ref[...]
    iota = jax.lax.broadcasted_iota(jnp.int32, (RB, L, VOCAB), 2)
    one_hot = (idx3[:, :, None] == iota).astype(jnp.bfloat16).reshape(R, VOCAB)
    mod = mod_ref[...].astype(jnp.bfloat16).reshape(R, MOD_IN)
    acc = jnp.dot(one_hot, wa_ref[...], preferred_element_type=jnp.float32)
    acc += jnp.dot(mod, wb_ref[...], preferred_element_type=jnp.float32)
    out_ref[...] = acc.reshape(RB, L, OUT_FEATURES)


def _tc_mod8_body(mod_ref, wb8_ref, out_ref):
    mod = mod_ref[...].astype(jnp.bfloat16).reshape(R, MOD_IN)
    out_ref[...] = jnp.dot(mod, wb8_ref[...], preferred_element_type=jnp.float32)


_sc_mesh = plsc.VectorSubcoreMesh(core_axis_name="c", subcore_axis_name="s")


@functools.partial(
    pl.kernel,
    mesh=_sc_mesh,
    compiler_params=pltpu.CompilerParams(use_tc_tiling_on_sc=False),
    out_type=jax.ShapeDtypeStruct((NTOK2, OUT_FEATURES), jnp.float32),
    scratch_types=[
        pltpu.VMEM((C,), jnp.int32),
        pltpu.VMEM((C, OUT_FEATURES), jnp.float32),
        pltpu.VMEM((C, MOD_OUT), jnp.float32),
        pltpu.SemaphoreType.DMA,
    ],
)
def _sc_gather(idx_hbm, tab_hbm, mod8_hbm, out_hbm, idx_v, rows_v, mod8_v, sem):
    wid = lax.axis_index("s") * NC + lax.axis_index("c")
    base = wid * PERW

    for ci in range(NCHUNK):
        off = base + ci * C
        pltpu.sync_copy(idx_hbm.at[pl.ds(off, C)], idx_v)
        pltpu.async_copy(tab_hbm.at[idx_v], rows_v, sem).wait()
        pltpu.sync_copy(mod8_hbm.at[pl.ds(off, C)],
                        rows_v.at[:, pl.ds(AA_DIM, MOD_OUT)])

        pltpu.sync_copy(rows_v, out_hbm.at[pl.ds(off, C)])


def kernel(aa_indices, mod_x, W_mod, aa_table):
    idx = aa_indices.astype(jnp.int32)

    wa = jnp.concatenate(
        [aa_table, jnp.zeros((VOCAB, MOD_OUT), jnp.float32)], axis=1
    ).astype(jnp.bfloat16)
    wb_top = jnp.concatenate(
        [jnp.zeros((K, AA_DIM), jnp.float32), jnp.eye(K, dtype=jnp.float32),
         jnp.zeros((K, MOD_OUT - K), jnp.float32)], axis=1)
    wb_bot = jnp.concatenate(
        [jnp.zeros((MOD_IN - K, AA_DIM + K), jnp.float32), W_mod], axis=1)
    wb = jnp.concatenate([wb_top, wb_bot], axis=0).astype(jnp.bfloat16)
    wb8_top = jnp.concatenate(
        [jnp.eye(K, dtype=jnp.float32), jnp.zeros((K, MOD_OUT - K), jnp.float32)],
        axis=1)
    wb8_bot = jnp.concatenate(
        [jnp.zeros((MOD_IN - K, K), jnp.float32), W_mod], axis=1)
    wb8 = jnp.concatenate([wb8_top, wb8_bot], axis=0).astype(jnp.bfloat16)

    tc_part = pl.pallas_call(
        _tc_full_body,
        grid=(B1 // RB,),
        in_specs=[
            pl.BlockSpec((RB, L), lambda i: (i, 0)),
            pl.BlockSpec((RB, L, MOD_IN), lambda i: (i, 0, 0)),
            pl.BlockSpec((VOCAB, OUT_FEATURES), lambda i: (0, 0)),
            pl.BlockSpec((MOD_IN, OUT_FEATURES), lambda i: (0, 0)),
        ],
        out_specs=pl.BlockSpec((RB, L, OUT_FEATURES), lambda i: (i, 0, 0)),
        out_shape=jax.ShapeDtypeStruct((B1, L, OUT_FEATURES), jnp.float32),
    )(idx[:B1], mod_x[:B1], wa, wb)

    mod8 = pl.pallas_call(
        _tc_mod8_body,
        grid=(B2 // RB,),
        in_specs=[
            pl.BlockSpec((RB, L, MOD_IN), lambda i: (i, 0, 0)),
            pl.BlockSpec((MOD_IN, MOD_OUT), lambda i: (0, 0)),
        ],
        out_specs=pl.BlockSpec((R, MOD_OUT), lambda i: (i, 0)),
        out_shape=jax.ShapeDtypeStruct((NTOK2, MOD_OUT), jnp.float32),
    )(mod_x[B1:], wb8)

    tab_pad = jnp.concatenate(
        [aa_table, jnp.zeros((VOCAB, MOD_OUT), jnp.float32)], axis=1)
    sc_part = _sc_gather(idx[B1:].reshape(NTOK2), tab_pad, mod8)

    return jnp.concatenate(
        [tc_part, sc_part.reshape(B2, L, OUT_FEATURES)], axis=0)


# speedup vs baseline: 3.4945x; 3.4945x over previous
import jax
import jax.numpy as jnp
from jax.experimental import pallas as pl

B, L = 4096, 64
MOD_IN = 109
K = 6
MOD_OUT = 8
OUT_FEATURES = 256
AA_DIM = OUT_FEATURES - MOD_OUT
VOCAB = 128

RB = 256
R = RB * L


def _body(idx_ref, mod_ref, wa_ref, wb_ref, out_ref):
    idx3 = idx_ref[...]
    iota = jax.lax.broadcasted_iota(jnp.int32, (RB, L, VOCAB), 2)
    one_hot = (idx3[:, :, None] == iota).astype(jnp.bfloat16).reshape(R, VOCAB)
    mod = mod_ref[...].astype(jnp.bfloat16).reshape(R, MOD_IN)
    acc = jnp.dot(one_hot, wa_ref[...], preferred_element_type=jnp.float32)
    acc += jnp.dot(mod, wb_ref[...], preferred_element_type=jnp.float32)
    out_ref[...] = acc.reshape(RB, L, OUT_FEATURES)


def kernel(aa_indices, mod_x, W_mod, aa_table):
    idx = aa_indices.astype(jnp.int32)

    wa = jnp.concatenate(
        [aa_table, jnp.zeros((VOCAB, MOD_OUT), jnp.float32)], axis=1
    ).astype(jnp.bfloat16)
    wb_top = jnp.concatenate(
        [jnp.zeros((K, AA_DIM), jnp.float32), jnp.eye(K, dtype=jnp.float32),
         jnp.zeros((K, OUT_FEATURES - AA_DIM - K), jnp.float32)], axis=1)
    wb_bot = jnp.concatenate(
        [jnp.zeros((MOD_IN - K, AA_DIM + K), jnp.float32), W_mod], axis=1)
    wb = jnp.concatenate([wb_top, wb_bot], axis=0).astype(jnp.bfloat16)

    return pl.pallas_call(
        _body,
        grid=(B // RB,),
        in_specs=[
            pl.BlockSpec((RB, L), lambda i: (i, 0)),
            pl.BlockSpec((RB, L, MOD_IN), lambda i: (i, 0, 0)),
            pl.BlockSpec((VOCAB, OUT_FEATURES), lambda i: (0, 0)),
            pl.BlockSpec((MOD_IN, OUT_FEATURES), lambda i: (0, 0)),
        ],
        out_specs=pl.BlockSpec((RB, L, OUT_FEATURES), lambda i: (i, 0, 0)),
        out_shape=jax.ShapeDtypeStruct((B, L, OUT_FEATURES), jnp.float32),
    )(idx, mod_x, wa, wb)
